# baseline (speedup 1.0000x reference)
import functools

import jax
import jax.numpy as jnp
from jax import lax
from jax.experimental import pallas as pl
from jax.experimental.pallas import tpu as pltpu
from jax.experimental.pallas import tpu_sc as plsc

_LANES = 16
_NW = 32
_CH = 2
_TC_BLK = 1024
_NPT0 = 528
_NPT1 = 112


def _tc_tables(x2p, W, b2):
    Cc, Npad = x2p.shape
    OUTc = W.shape[0]

    def body(x_ref, w_ref, b_ref, a_ref, t_ref):
        xb = x_ref[...]
        wd = w_ref[:, :Cc] - w_ref[:, Cc:]
        w2 = w_ref[:, Cc:]
        a = lax.dot_general(xb, wd, (((0,), (1,)), ((), ())),
                            precision=lax.Precision.HIGHEST,
                            preferred_element_type=jnp.float32)
        a_ref[...] = a + b_ref[...]
        t_ref[...] = lax.dot_general(xb, w2, (((0,), (1,)), ((), ())),
                                     precision=lax.Precision.HIGHEST,
                                     preferred_element_type=jnp.float32)

    return pl.pallas_call(
        body,
        grid=(Npad // _TC_BLK,),
        in_specs=[
            pl.BlockSpec((Cc, _TC_BLK), lambda i: (0, i)),
            pl.BlockSpec((OUTc, 2 * Cc), lambda i: (0, 0)),
            pl.BlockSpec((1, OUTc), lambda i: (0, 0)),
        ],
        out_specs=[
            pl.BlockSpec((_TC_BLK, OUTc), lambda i: (i, 0)),
            pl.BlockSpec((_TC_BLK, OUTc), lambda i: (i, 0)),
        ],
        out_shape=[
            jax.ShapeDtypeStruct((Npad, OUTc), jnp.float32),
            jax.ShapeDtypeStruct((Npad, OUTc), jnp.float32),
        ],
    )(x2p, W, b2)


def _sc_edge_max(A, T, ii, jj, px, py, pz, Npad, OUTc, Kc):
    CHK = _CH * Kc
    NV = OUTc // _LANES
    NLOC = max(_NPT0, _NPT1) // _CH + 2
    mesh = plsc.VectorSubcoreMesh(core_axis_name="c", subcore_axis_name="s")

    @functools.partial(
        pl.kernel,
        mesh=mesh,
        compiler_params=pltpu.CompilerParams(needs_layout_passes=False),
        out_type=jax.ShapeDtypeStruct((Npad, OUTc), jnp.float32),
        scratch_types=[
            pltpu.VMEM((NLOC, CHK), jnp.int32),
            pltpu.VMEM((NLOC, CHK), jnp.int32),
            pltpu.VMEM((Npad,), jnp.float32),
            pltpu.VMEM((Npad,), jnp.float32),
            pltpu.VMEM((Npad,), jnp.float32),
            pltpu.VMEM((CHK, OUTc), jnp.float32),
            pltpu.VMEM((CHK, OUTc), jnp.float32),
            pltpu.VMEM((_CH, OUTc), jnp.float32),
            pltpu.SemaphoreType.DMA,
            pltpu.SemaphoreType.DMA,
            pltpu.SemaphoreType.DMA,
            pltpu.SemaphoreType.DMA,
        ],
    )
    def sck(a_hbm, t_hbm, ii_hbm, jj_hbm, px_hbm, py_hbm, pz_hbm, out_hbm,
            ii_v, jj_v, px_v, py_v, pz_v, rb0_v, rb1_v, o_v,
            sa0, sa1, st0, st1):
        cid = lax.axis_index("c")
        sid = lax.axis_index("s")
        wid = sid * 2 + cid
        base = sid * (_NPT0 + _NPT1) + cid * _NPT0
        npairs = lax.select(cid == 0, (_NPT0 // _CH) // 2,
                            (_NPT1 // _CH) // 2)
        pltpu.sync_copy(ii_hbm.at[wid], ii_v)
        pltpu.sync_copy(jj_hbm.at[wid], jj_v)
        pltpu.sync_copy(px_hbm, px_v)
        pltpu.sync_copy(py_hbm, py_v)
        pltpu.sync_copy(pz_hbm, pz_v)

        iota = lax.broadcasted_iota(jnp.int32, (_LANES,), 0)
        bufs = (rb0_v, rb1_v)
        sas = (sa0, sa1)
        sts = (st0, st1)

        def start_a(ch, b):
            return pltpu.async_copy(a_hbm.at[ii_v.at[ch]], bufs[b], sas[b])

        def start_t(ch, b):
            return pltpu.async_copy(
                t_hbm.at[jj_v.at[ch]], bufs[b], sts[b], add=True)

        def wait_a(b):
            pltpu.make_async_copy(a_hbm.at[ii_v.at[0]], bufs[b],
                                  sas[b]).wait()

        def wait_t(b):
            pltpu.make_async_copy(a_hbm.at[ii_v.at[0]], bufs[b],
                                  sts[b]).wait()

        def do_chunk(ch, b):
            wait_a(1 - b)
            start_t(ch + 1, 1 - b)
            chv = jnp.full((_LANES,), 0, jnp.int32) + ch
            s_regs = []
            for g in range(CHK // _LANES):
                iv = plsc.load_gather(ii_v, [chv, iota + g * _LANES])
                jv = plsc.load_gather(jj_v, [chv, iota + g * _LANES])
                dx = plsc.load_gather(px_v, [jv]) - plsc.load_gather(px_v, [iv])
                dy = plsc.load_gather(py_v, [jv]) - plsc.load_gather(py_v, [iv])
                dz = plsc.load_gather(pz_v, [jv]) - plsc.load_gather(pz_v, [iv])
                d2 = dx * dx + dy * dy + dz * dz
                ib = plsc.bitcast(d2, jnp.int32)
                y = plsc.bitcast(
                    jnp.full((_LANES,), 0x5F3759DF, jnp.int32)
                    - jnp.right_shift(ib, 1), jnp.float32)
                for _ in range(3):
                    y = y * (1.5 - 0.5 * d2 * y * y)
                dis = d2 * y
                s_regs.append(2.0 / (1.0 + jnp.exp(dis)))
            wait_t(b)
            rb = bufs[b]
            for nn in range(_CH):
                acc = [jnp.full((_LANES,), 0.0, jnp.float32)
                       for _ in range(NV)]
                for k in range(Kc):
                    e = nn * Kc + k
                    sv = s_regs[e // _LANES].at[
                        jnp.full((_LANES,), e % _LANES, jnp.int32)
                    ].get(mode="promise_in_bounds")
                    for c in range(NV):
                        f = jnp.maximum(rb[e, pl.ds(c * _LANES, _LANES)],
                                        0.0) * sv
                        acc[c] = jnp.maximum(acc[c], f)
                for c in range(NV):
                    o_v[nn, pl.ds(c * _LANES, _LANES)] = acc[c]
            pltpu.sync_copy(o_v, out_hbm.at[pl.ds(base + ch * _CH, _CH)])
            start_a(ch + 2, b)

        start_a(0, 0)
        wait_a(0)
        start_t(0, 0)
        start_a(1, 1)

        def pair(t, carry):
            do_chunk(2 * t, 0)
            do_chunk(2 * t + 1, 1)
            return carry

        lax.fori_loop(0, npairs, pair, 0)
        wait_t(0)
        wait_a(1)

    return sck(A, T, ii, jj, px, py, pz)


def kernel(x, edge_index, pos, W, b):
    _, Cc, Nn, _ = x.shape
    Kc = edge_index.shape[-1]
    OUTc = W.shape[0]
    align = max(_TC_BLK, 16 * (_NPT0 + _NPT1))
    Npad = ((Nn + align - 1) // align) * align

    x2p = jnp.pad(x[0, :, :, 0], ((0, 0), (0, Npad - Nn)))
    A, T = _tc_tables(x2p, W, b.reshape(1, OUTc))

    ei = edge_index.astype(jnp.int32)
    pad_n = ((0, Npad - Nn), (0, 0))
    nloc = max(_NPT0, _NPT1) // _CH + 2
    ngch = Npad // _CH
    iig = jnp.pad(jnp.pad(ei[1, 0], pad_n).reshape(ngch, _CH * Kc),
                  ((0, nloc), (0, 0)))
    jjg = jnp.pad(jnp.pad(ei[0, 0], pad_n).reshape(ngch, _CH * Kc),
                  ((0, nloc), (0, 0)))
    starts = jnp.asarray(
        [s * (_NPT0 + _NPT1) // _CH + c * _NPT0 // _CH
         for s in range(16) for c in range(2)], dtype=jnp.int32)
    rows_idx = starts[:, None] + jnp.arange(nloc, dtype=jnp.int32)[None, :]
    ii = jnp.take(iig, rows_idx, axis=0)
    jj = jnp.take(jjg, rows_idx, axis=0)
    p3p = jnp.pad(pos[0, :, :, 0], ((0, 0), (0, Npad - Nn)))

    rows = _sc_edge_max(A, T, ii, jj, p3p[0], p3p[1], p3p[2],
                        Npad, OUTc, Kc)
    max_value = rows[:Nn].T[None, :, :, None]
    return (max_value, edge_index, pos)


# speedup vs baseline: 1.0525x; 1.0525x over previous
import functools

import jax
import jax.numpy as jnp
from jax import lax
from jax.experimental import pallas as pl
from jax.experimental.pallas import tpu as pltpu
from jax.experimental.pallas import tpu_sc as plsc

_LANES = 16
_NW = 32
_CH = 2
_TC_BLK = 1024
_NPT0 = 496
_NPT1 = 144


def _tc_tables(x2p, W, b2):
    Cc, Npad = x2p.shape
    OUTc = W.shape[0]

    def body(x_ref, w_ref, b_ref, a_ref, t_ref):
        xb = x_ref[...]
        wd = w_ref[:, :Cc] - w_ref[:, Cc:]
        w2 = w_ref[:, Cc:]
        a = lax.dot_general(xb, wd, (((0,), (1,)), ((), ())),
                            precision=lax.Precision.HIGHEST,
                            preferred_element_type=jnp.float32)
        a_ref[...] = a + b_ref[...]
        t_ref[...] = lax.dot_general(xb, w2, (((0,), (1,)), ((), ())),
                                     precision=lax.Precision.HIGHEST,
                                     preferred_element_type=jnp.float32)

    return pl.pallas_call(
        body,
        grid=(Npad // _TC_BLK,),
        in_specs=[
            pl.BlockSpec((Cc, _TC_BLK), lambda i: (0, i)),
            pl.BlockSpec((OUTc, 2 * Cc), lambda i: (0, 0)),
            pl.BlockSpec((1, OUTc), lambda i: (0, 0)),
        ],
        out_specs=[
            pl.BlockSpec((_TC_BLK, OUTc), lambda i: (i, 0)),
            pl.BlockSpec((_TC_BLK, OUTc), lambda i: (i, 0)),
        ],
        out_shape=[
            jax.ShapeDtypeStruct((Npad, OUTc), jnp.float32),
            jax.ShapeDtypeStruct((Npad, OUTc), jnp.float32),
        ],
    )(x2p, W, b2)


def _sc_edge_max(A, T, ii, jj, px, py, pz, Npad, OUTc, Kc):
    CHK = _CH * Kc
    NV = OUTc // _LANES
    NLOC = max(_NPT0, _NPT1) // _CH + 2
    mesh = plsc.VectorSubcoreMesh(core_axis_name="c", subcore_axis_name="s")

    @functools.partial(
        pl.kernel,
        mesh=mesh,
        compiler_params=pltpu.CompilerParams(needs_layout_passes=False),
        out_type=jax.ShapeDtypeStruct((Npad, OUTc), jnp.float32),
        scratch_types=[
            pltpu.VMEM((NLOC, CHK), jnp.int32),
            pltpu.VMEM((NLOC, CHK), jnp.int32),
            pltpu.VMEM((Npad,), jnp.float32),
            pltpu.VMEM((Npad,), jnp.float32),
            pltpu.VMEM((Npad,), jnp.float32),
            pltpu.VMEM((CHK, OUTc), jnp.float32),
            pltpu.VMEM((CHK, OUTc), jnp.float32),
            pltpu.VMEM((_CH, OUTc), jnp.float32),
            pltpu.SemaphoreType.DMA,
            pltpu.SemaphoreType.DMA,
            pltpu.SemaphoreType.DMA,
            pltpu.SemaphoreType.DMA,
        ],
    )
    def sck(a_hbm, t_hbm, ii_hbm, jj_hbm, px_hbm, py_hbm, pz_hbm, out_hbm,
            ii_v, jj_v, px_v, py_v, pz_v, rb0_v, rb1_v, o_v,
            sa0, sa1, st0, st1):
        cid = lax.axis_index("c")
        sid = lax.axis_index("s")
        wid = sid * 2 + cid
        base = sid * (_NPT0 + _NPT1) + cid * _NPT0
        npairs = lax.select(cid == 0, (_NPT0 // _CH) // 2,
                            (_NPT1 // _CH) // 2)
        pltpu.sync_copy(ii_hbm.at[wid], ii_v)
        pltpu.sync_copy(jj_hbm.at[wid], jj_v)
        pltpu.sync_copy(px_hbm, px_v)
        pltpu.sync_copy(py_hbm, py_v)
        pltpu.sync_copy(pz_hbm, pz_v)

        iota = lax.broadcasted_iota(jnp.int32, (_LANES,), 0)
        bufs = (rb0_v, rb1_v)
        sas = (sa0, sa1)
        sts = (st0, st1)

        def start_a(ch, b):
            return pltpu.async_copy(a_hbm.at[ii_v.at[ch]], bufs[b], sas[b])

        def start_t(ch, b):
            return pltpu.async_copy(
                t_hbm.at[jj_v.at[ch]], bufs[b], sts[b], add=True)

        def wait_a(b):
            pltpu.make_async_copy(a_hbm.at[ii_v.at[0]], bufs[b],
                                  sas[b]).wait()

        def wait_t(b):
            pltpu.make_async_copy(a_hbm.at[ii_v.at[0]], bufs[b],
                                  sts[b]).wait()

        def do_chunk(ch, b):
            wait_a(1 - b)
            start_t(ch + 1, 1 - b)
            chv = jnp.full((_LANES,), 0, jnp.int32) + ch
            s_regs = []
            for g in range(CHK // _LANES):
                iv = plsc.load_gather(ii_v, [chv, iota + g * _LANES])
                jv = plsc.load_gather(jj_v, [chv, iota + g * _LANES])
                dx = plsc.load_gather(px_v, [jv]) - plsc.load_gather(px_v, [iv])
                dy = plsc.load_gather(py_v, [jv]) - plsc.load_gather(py_v, [iv])
                dz = plsc.load_gather(pz_v, [jv]) - plsc.load_gather(pz_v, [iv])
                d2 = dx * dx + dy * dy + dz * dz
                ib = plsc.bitcast(d2, jnp.int32)
                y = plsc.bitcast(
                    jnp.full((_LANES,), 0x5F3759DF, jnp.int32)
                    - jnp.right_shift(ib, 1), jnp.float32)
                for _ in range(3):
                    y = y * (1.5 - 0.5 * d2 * y * y)
                dis = d2 * y
                s_regs.append(2.0 / (1.0 + jnp.exp(dis)))
            wait_t(b)
            rb = bufs[b]
            for nn in range(_CH):
                acc = [jnp.full((_LANES,), 0.0, jnp.float32)
                       for _ in range(NV)]
                for k in range(Kc):
                    e = nn * Kc + k
                    sv = s_regs[e // _LANES].at[
                        jnp.full((_LANES,), e % _LANES, jnp.int32)
                    ].get(mode="promise_in_bounds")
                    for c in range(NV):
                        f = jnp.maximum(rb[e, pl.ds(c * _LANES, _LANES)],
                                        0.0) * sv
                        acc[c] = jnp.maximum(acc[c], f)
                for c in range(NV):
                    o_v[nn, pl.ds(c * _LANES, _LANES)] = acc[c]
            pltpu.sync_copy(o_v, out_hbm.at[pl.ds(base + ch * _CH, _CH)])
            start_a(ch + 2, b)

        start_a(0, 0)
        wait_a(0)
        start_t(0, 0)
        start_a(1, 1)

        def pair(t, carry):
            do_chunk(2 * t, 0)
            do_chunk(2 * t + 1, 1)
            return carry

        lax.fori_loop(0, npairs, pair, 0)
        wait_t(0)
        wait_a(1)

    return sck(A, T, ii, jj, px, py, pz)


def kernel(x, edge_index, pos, W, b):
    _, Cc, Nn, _ = x.shape
    Kc = edge_index.shape[-1]
    OUTc = W.shape[0]
    align = max(_TC_BLK, 16 * (_NPT0 + _NPT1))
    Npad = ((Nn + align - 1) // align) * align

    x2p = jnp.pad(x[0, :, :, 0], ((0, 0), (0, Npad - Nn)))
    A, T = _tc_tables(x2p, W, b.reshape(1, OUTc))

    ei = edge_index.astype(jnp.int32)
    pad_n = ((0, Npad - Nn), (0, 0))
    nloc = max(_NPT0, _NPT1) // _CH + 2
    ngch = Npad // _CH
    iig = jnp.pad(jnp.pad(ei[1, 0], pad_n).reshape(ngch, _CH * Kc),
                  ((0, nloc), (0, 0)))
    jjg = jnp.pad(jnp.pad(ei[0, 0], pad_n).reshape(ngch, _CH * Kc),
                  ((0, nloc), (0, 0)))
    starts = jnp.asarray(
        [s * (_NPT0 + _NPT1) // _CH + c * _NPT0 // _CH
         for s in range(16) for c in range(2)], dtype=jnp.int32)
    rows_idx = starts[:, None] + jnp.arange(nloc, dtype=jnp.int32)[None, :]
    ii = jnp.take(iig, rows_idx, axis=0)
    jj = jnp.take(jjg, rows_idx, axis=0)
    p3p = jnp.pad(pos[0, :, :, 0], ((0, 0), (0, Npad - Nn)))

    rows = _sc_edge_max(A, T, ii, jj, p3p[0], p3p[1], p3p[2],
                        Npad, OUTc, Kc)
    max_value = rows[:Nn].T[None, :, :, None]
    return (max_value, edge_index, pos)


# speedup vs baseline: 1.2357x; 1.1740x over previous
import functools

import jax
import jax.numpy as jnp
from jax import lax
from jax.experimental import pallas as pl
from jax.experimental.pallas import tpu as pltpu
from jax.experimental.pallas import tpu_sc as plsc

_LANES = 16
_NW = 32
_CH = 2
_TC_BLK = 1024
_NPT0 = 464
_NPT1 = 176


def _tc_tables(x2p, W, b2):
    Cc, Npad = x2p.shape
    OUTc = W.shape[0]

    def body(x_ref, w_ref, b_ref, a_ref, t_ref):
        xb = x_ref[...]
        wd = w_ref[:, :Cc] - w_ref[:, Cc:]
        w2 = w_ref[:, Cc:]
        a = lax.dot_general(xb, wd, (((0,), (1,)), ((), ())),
                            precision=lax.Precision.HIGHEST,
                            preferred_element_type=jnp.float32)
        a_ref[...] = a + b_ref[...]
        t_ref[...] = lax.dot_general(xb, w2, (((0,), (1,)), ((), ())),
                                     precision=lax.Precision.HIGHEST,
                                     preferred_element_type=jnp.float32)

    return pl.pallas_call(
        body,
        grid=(Npad // _TC_BLK,),
        in_specs=[
            pl.BlockSpec((Cc, _TC_BLK), lambda i: (0, i)),
            pl.BlockSpec((OUTc, 2 * Cc), lambda i: (0, 0)),
            pl.BlockSpec((1, OUTc), lambda i: (0, 0)),
        ],
        out_specs=[
            pl.BlockSpec((_TC_BLK, OUTc), lambda i: (i, 0)),
            pl.BlockSpec((_TC_BLK, OUTc), lambda i: (i, 0)),
        ],
        out_shape=[
            jax.ShapeDtypeStruct((Npad, OUTc), jnp.float32),
            jax.ShapeDtypeStruct((Npad, OUTc), jnp.float32),
        ],
    )(x2p, W, b2)


def _sc_edge_max(A, T, ii, jj, px, py, pz, Npad, OUTc, Kc):
    CHK = _CH * Kc
    NV = OUTc // _LANES
    NLOC = max(_NPT0, _NPT1) // _CH + 2
    mesh = plsc.VectorSubcoreMesh(core_axis_name="c", subcore_axis_name="s")

    @functools.partial(
        pl.kernel,
        mesh=mesh,
        compiler_params=pltpu.CompilerParams(needs_layout_passes=False),
        out_type=jax.ShapeDtypeStruct((Npad, OUTc), jnp.float32),
        scratch_types=[
            pltpu.VMEM((NLOC, CHK), jnp.int32),
            pltpu.VMEM((NLOC, CHK), jnp.int32),
            pltpu.VMEM((Npad,), jnp.float32),
            pltpu.VMEM((Npad,), jnp.float32),
            pltpu.VMEM((Npad,), jnp.float32),
            pltpu.VMEM((CHK, OUTc), jnp.float32),
            pltpu.VMEM((CHK, OUTc), jnp.float32),
            pltpu.VMEM((_CH, OUTc), jnp.float32),
            pltpu.SemaphoreType.DMA,
            pltpu.SemaphoreType.DMA,
            pltpu.SemaphoreType.DMA,
            pltpu.SemaphoreType.DMA,
        ],
    )
    def sck(a_hbm, t_hbm, ii_hbm, jj_hbm, px_hbm, py_hbm, pz_hbm, out_hbm,
            ii_v, jj_v, px_v, py_v, pz_v, rb0_v, rb1_v, o_v,
            sa0, sa1, st0, st1):
        cid = lax.axis_index("c")
        sid = lax.axis_index("s")
        wid = sid * 2 + cid
        base = sid * (_NPT0 + _NPT1) + cid * _NPT0
        npairs = lax.select(cid == 0, (_NPT0 // _CH) // 2,
                            (_NPT1 // _CH) // 2)
        pltpu.sync_copy(ii_hbm.at[wid], ii_v)
        pltpu.sync_copy(jj_hbm.at[wid], jj_v)
        pltpu.sync_copy(px_hbm, px_v)
        pltpu.sync_copy(py_hbm, py_v)
        pltpu.sync_copy(pz_hbm, pz_v)

        iota = lax.broadcasted_iota(jnp.int32, (_LANES,), 0)
        bufs = (rb0_v, rb1_v)
        sas = (sa0, sa1)
        sts = (st0, st1)

        def start_a(ch, b):
            return pltpu.async_copy(a_hbm.at[ii_v.at[ch]], bufs[b], sas[b])

        def start_t(ch, b):
            return pltpu.async_copy(
                t_hbm.at[jj_v.at[ch]], bufs[b], sts[b], add=True)

        def wait_a(b):
            pltpu.make_async_copy(a_hbm.at[ii_v.at[0]], bufs[b],
                                  sas[b]).wait()

        def wait_t(b):
            pltpu.make_async_copy(a_hbm.at[ii_v.at[0]], bufs[b],
                                  sts[b]).wait()

        def do_chunk(ch, b):
            wait_a(1 - b)
            start_t(ch + 1, 1 - b)
            chv = jnp.full((_LANES,), 0, jnp.int32) + ch
            s_regs = []
            for g in range(CHK // _LANES):
                iv = plsc.load_gather(ii_v, [chv, iota + g * _LANES])
                jv = plsc.load_gather(jj_v, [chv, iota + g * _LANES])
                dx = plsc.load_gather(px_v, [jv]) - plsc.load_gather(px_v, [iv])
                dy = plsc.load_gather(py_v, [jv]) - plsc.load_gather(py_v, [iv])
                dz = plsc.load_gather(pz_v, [jv]) - plsc.load_gather(pz_v, [iv])
                d2 = dx * dx + dy * dy + dz * dz
                ib = plsc.bitcast(d2, jnp.int32)
                y = plsc.bitcast(
                    jnp.full((_LANES,), 0x5F3759DF, jnp.int32)
                    - jnp.right_shift(ib, 1), jnp.float32)
                for _ in range(3):
                    y = y * (1.5 - 0.5 * d2 * y * y)
                dis = d2 * y
                s_regs.append(2.0 / (1.0 + jnp.exp(dis)))
            wait_t(b)
            rb = bufs[b]
            for nn in range(_CH):
                acc = [jnp.full((_LANES,), 0.0, jnp.float32)
                       for _ in range(NV)]
                for k in range(Kc):
                    e = nn * Kc + k
                    sv = s_regs[e // _LANES].at[
                        jnp.full((_LANES,), e % _LANES, jnp.int32)
                    ].get(mode="promise_in_bounds")
                    for c in range(NV):
                        f = jnp.maximum(rb[e, pl.ds(c * _LANES, _LANES)],
                                        0.0) * sv
                        acc[c] = jnp.maximum(acc[c], f)
                for c in range(NV):
                    o_v[nn, pl.ds(c * _LANES, _LANES)] = acc[c]
            pltpu.sync_copy(o_v, out_hbm.at[pl.ds(base + ch * _CH, _CH)])
            start_a(ch + 2, b)

        start_a(0, 0)
        wait_a(0)
        start_t(0, 0)
        start_a(1, 1)

        def pair(t, carry):
            do_chunk(2 * t, 0)
            do_chunk(2 * t + 1, 1)
            return carry

        lax.fori_loop(0, npairs, pair, 0)
        wait_t(0)
        wait_a(1)

    return sck(A, T, ii, jj, px, py, pz)


def kernel(x, edge_index, pos, W, b):
    _, Cc, Nn, _ = x.shape
    Kc = edge_index.shape[-1]
    OUTc = W.shape[0]
    align = max(_TC_BLK, 16 * (_NPT0 + _NPT1))
    Npad = ((Nn + align - 1) // align) * align

    x2p = jnp.pad(x[0, :, :, 0], ((0, 0), (0, Npad - Nn)))
    A, T = _tc_tables(x2p, W, b.reshape(1, OUTc))

    ei = edge_index.astype(jnp.int32)
    pad_n = ((0, Npad - Nn), (0, 0))
    nloc = max(_NPT0, _NPT1) // _CH + 2
    ngch = Npad // _CH
    iig = jnp.pad(jnp.pad(ei[1, 0], pad_n).reshape(ngch, _CH * Kc),
                  ((0, nloc), (0, 0)))
    jjg = jnp.pad(jnp.pad(ei[0, 0], pad_n).reshape(ngch, _CH * Kc),
                  ((0, nloc), (0, 0)))
    starts = jnp.asarray(
        [s * (_NPT0 + _NPT1) // _CH + c * _NPT0 // _CH
         for s in range(16) for c in range(2)], dtype=jnp.int32)
    rows_idx = starts[:, None] + jnp.arange(nloc, dtype=jnp.int32)[None, :]
    ii = jnp.take(iig, rows_idx, axis=0)
    jj = jnp.take(jjg, rows_idx, axis=0)
    p3p = jnp.pad(pos[0, :, :, 0], ((0, 0), (0, Npad - Nn)))

    rows = _sc_edge_max(A, T, ii, jj, p3p[0], p3p[1], p3p[2],
                        Npad, OUTc, Kc)
    max_value = rows[:Nn].T[None, :, :, None]
    return (max_value, edge_index, pos)
